# baseline (speedup 1.0000x reference)
import functools

import jax
import jax.numpy as jnp
from jax import lax
from jax.experimental import pallas as pl
from jax.experimental.pallas import tpu as pltpu
from jax.experimental.pallas import tpu_sc as plsc

NC = 2
NS = 16
NW = NC * NS
L = 16

BATCH = 16384
D = 64
DP = 2 * D
B_PER_W = BATCH // NW
CHUNK = 128
NCHUNK = B_PER_W // CHUNK
NGROUP = CHUNK // L


def _sc_body(h_hbm, r_hbm, t_hbm, ere_hbm, eim_hbm, rre_hbm, rim_hbm,
             out_hbm, idx_h, idx_r, idx_t, tl_h, tl_r, tl_t,
             hr_b, hi_b, tr_b, ti_b, rr_b, ri_b, out_v, sem):
    wid = lax.axis_index("s") * NC + lax.axis_index("c")
    base = wid * B_PER_W

    rows0 = lax.iota(jnp.int32, L)

    for c in range(NCHUNK):
        off = base + c * CHUNK
        pltpu.sync_copy(h_hbm.at[pl.ds(off, CHUNK)], idx_h)
        pltpu.sync_copy(r_hbm.at[pl.ds(off, CHUNK)], idx_r)
        pltpu.sync_copy(t_hbm.at[pl.ds(off, CHUNK)], idx_t)

        for q in range(NGROUP):
            qs = pl.ds(q * L, L)
            tl_h[qs] = idx_h[qs] >> 1
            tl_r[qs] = idx_r[qs] >> 1
            tl_t[qs] = idx_t[qs] >> 1

        copies = [
            pltpu.async_copy(ere_hbm.at[tl_h], hr_b, sem),
            pltpu.async_copy(eim_hbm.at[tl_h], hi_b, sem),
            pltpu.async_copy(ere_hbm.at[tl_t], tr_b, sem),
            pltpu.async_copy(eim_hbm.at[tl_t], ti_b, sem),
            pltpu.async_copy(rre_hbm.at[tl_r], rr_b, sem),
            pltpu.async_copy(rim_hbm.at[tl_r], ri_b, sem),
        ]
        for cp in copies:
            cp.wait()

        for g in range(NGROUP):
            gs = pl.ds(g * L, L)
            rows = rows0 + g * L
            ch = (idx_h[gs] & 1) * D
            cr = (idx_r[gs] & 1) * D
            ct = (idx_t[gs] & 1) * D

            def dim_step(d, acc):
                dv = jnp.full((L,), d, jnp.int32)
                hr = plsc.load_gather(hr_b, [rows, ch + dv])
                hi = plsc.load_gather(hi_b, [rows, ch + dv])
                tr = plsc.load_gather(tr_b, [rows, ct + dv])
                ti = plsc.load_gather(ti_b, [rows, ct + dv])
                rr = plsc.load_gather(rr_b, [rows, cr + dv])
                ri = plsc.load_gather(ri_b, [rows, cr + dv])
                a = hr * rr - hi * ri
                b = hi * rr + hr * ri
                return acc + a * tr + b * ti

            acc = lax.fori_loop(0, D, dim_step, jnp.zeros((L,), jnp.float32))
            out_v[pl.ds(c * CHUNK + g * L, L)] = acc

    pltpu.sync_copy(out_v, out_hbm.at[pl.ds(base, B_PER_W)])


@jax.jit
def _complex_score(h, r, t, entity_re, entity_im, relation_re, relation_im):
    ere2 = entity_re.reshape(-1, DP)
    eim2 = entity_im.reshape(-1, DP)
    rre2 = relation_re.reshape(-1, DP)
    rim2 = relation_im.reshape(-1, DP)
    mesh = plsc.VectorSubcoreMesh(core_axis_name="c", subcore_axis_name="s")
    run = functools.partial(
        pl.kernel,
        out_type=jax.ShapeDtypeStruct((BATCH,), jnp.float32),
        mesh=mesh,
        compiler_params=pltpu.CompilerParams(needs_layout_passes=False),
        scratch_types=[
            pltpu.VMEM((CHUNK,), jnp.int32),
            pltpu.VMEM((CHUNK,), jnp.int32),
            pltpu.VMEM((CHUNK,), jnp.int32),
            pltpu.VMEM((CHUNK,), jnp.int32),
            pltpu.VMEM((CHUNK,), jnp.int32),
            pltpu.VMEM((CHUNK,), jnp.int32),
            pltpu.VMEM((CHUNK, DP), jnp.float32),
            pltpu.VMEM((CHUNK, DP), jnp.float32),
            pltpu.VMEM((CHUNK, DP), jnp.float32),
            pltpu.VMEM((CHUNK, DP), jnp.float32),
            pltpu.VMEM((CHUNK, DP), jnp.float32),
            pltpu.VMEM((CHUNK, DP), jnp.float32),
            pltpu.VMEM((B_PER_W,), jnp.float32),
            pltpu.SemaphoreType.DMA,
        ],
    )(_sc_body)
    return run(h, r, t, ere2, eim2, rre2, rim2)


def kernel(h, r, t, entity_re, entity_im, relation_re, relation_im):
    return _complex_score(h.astype(jnp.int32), r.astype(jnp.int32),
                          t.astype(jnp.int32), entity_re, entity_im,
                          relation_re, relation_im)


# speedup vs baseline: 1.4690x; 1.4690x over previous
import functools

import jax
import jax.numpy as jnp
from jax import lax
from jax.experimental import pallas as pl
from jax.experimental.pallas import tpu as pltpu
from jax.experimental.pallas import tpu_sc as plsc

NC = 2
NS = 16
NW = NC * NS
L = 16

BATCH = 16384
D = 64
SL = 8
B_PER_W = BATCH // NW
CHUNK = 128
NCHUNK = B_PER_W // CHUNK
NGROUP = CHUNK // L

NUM_REL = 1000
REL_SLABS = NUM_REL // SL
SLABS_PER_TILE = (REL_SLABS + NS - 1) // NS


def _sc_body(h_hbm, r_hbm, t_hbm, ere_hbm, eim_hbm, rre_hbm, rim_hbm,
             out_hbm, idx_h, idx_r, idx_t,
             hr_b, hi_b, tr_b, ti_b, rr_b, ri_b, out_v,
             rre_sp, rim_sp, sem_h, sem_i, sem_t, sem_j, sem_r):
    sid = lax.axis_index("s")
    wid = sid * NC + lax.axis_index("c")
    base = wid * B_PER_W

    rows0 = lax.iota(jnp.int32, L)

    rre_v = rre_hbm.reshape(REL_SLABS, SL, D)
    rim_v = rim_hbm.reshape(REL_SLABS, SL, D)
    rre_sp3 = rre_sp.reshape(REL_SLABS, SL, D)
    rim_sp3 = rim_sp.reshape(REL_SLABS, SL, D)
    for q in range(SLABS_PER_TILE):
        k = sid * SLABS_PER_TILE + q

        @pl.when(k < REL_SLABS)
        def _():
            pltpu.async_copy(rre_v.at[pl.ds(k, 1)], rre_sp3.at[pl.ds(k, 1)],
                             sem_r).wait()
            pltpu.async_copy(rim_v.at[pl.ds(k, 1)], rim_sp3.at[pl.ds(k, 1)],
                             sem_r).wait()

    plsc.subcore_barrier()

    for c in range(NCHUNK):
        off = base + c * CHUNK
        pltpu.sync_copy(h_hbm.at[pl.ds(off, CHUNK)], idx_h)
        pltpu.sync_copy(r_hbm.at[pl.ds(off, CHUNK)], idx_r)
        pltpu.sync_copy(t_hbm.at[pl.ds(off, CHUNK)], idx_t)

        rel_copies = [
            pltpu.async_copy(rre_sp.at[idx_r], rr_b, sem_r),
            pltpu.async_copy(rim_sp.at[idx_r], ri_b, sem_r),
        ]

        def group_dma(g, carry):
            gs = pl.ds(g * L, L)
            ihv = idx_h[gs]
            itv = idx_t[gs]
            for j in range(L):
                m = rows0 == j
                ih = jnp.sum(jnp.where(m, ihv, 0))
                it = jnp.sum(jnp.where(m, itv, 0))
                dst = pl.ds(g * L + j, 1)
                pltpu.async_copy(ere_hbm.at[pl.ds(ih, 1)], hr_b.at[dst],
                                 sem_h)
                pltpu.async_copy(eim_hbm.at[pl.ds(ih, 1)], hi_b.at[dst],
                                 sem_i)
                pltpu.async_copy(ere_hbm.at[pl.ds(it, 1)], tr_b.at[dst],
                                 sem_t)
                pltpu.async_copy(eim_hbm.at[pl.ds(it, 1)], ti_b.at[dst],
                                 sem_j)
            return carry

        lax.fori_loop(0, NGROUP, group_dma, 0)

        pltpu.make_async_copy(ere_hbm.at[pl.ds(0, CHUNK)], hr_b, sem_h).wait()
        pltpu.make_async_copy(ere_hbm.at[pl.ds(0, CHUNK)], hi_b, sem_i).wait()
        pltpu.make_async_copy(ere_hbm.at[pl.ds(0, CHUNK)], tr_b, sem_t).wait()
        pltpu.make_async_copy(ere_hbm.at[pl.ds(0, CHUNK)], ti_b, sem_j).wait()
        for cp in rel_copies:
            cp.wait()

        for g in range(NGROUP):
            rows = rows0 + g * L

            def dim_step(d, acc):
                dv = jnp.full((L,), d, jnp.int32)
                hr = plsc.load_gather(hr_b, [rows, dv])
                hi = plsc.load_gather(hi_b, [rows, dv])
                tr = plsc.load_gather(tr_b, [rows, dv])
                ti = plsc.load_gather(ti_b, [rows, dv])
                rr = plsc.load_gather(rr_b, [rows, dv])
                ri = plsc.load_gather(ri_b, [rows, dv])
                a = hr * rr - hi * ri
                b = hi * rr + hr * ri
                return acc + a * tr + b * ti

            acc = lax.fori_loop(0, D, dim_step, jnp.zeros((L,), jnp.float32))
            out_v[pl.ds(c * CHUNK + g * L, L)] = acc

    pltpu.sync_copy(out_v, out_hbm.at[pl.ds(base, B_PER_W)])


@jax.jit
def _complex_score(h, r, t, entity_re, entity_im, relation_re, relation_im):
    mesh = plsc.VectorSubcoreMesh(core_axis_name="c", subcore_axis_name="s")
    run = functools.partial(
        pl.kernel,
        out_type=jax.ShapeDtypeStruct((BATCH,), jnp.float32),
        mesh=mesh,
        compiler_params=pltpu.CompilerParams(needs_layout_passes=False),
        scratch_types=[
            pltpu.VMEM((CHUNK,), jnp.int32),
            pltpu.VMEM((CHUNK,), jnp.int32),
            pltpu.VMEM((CHUNK,), jnp.int32),
            pltpu.VMEM((CHUNK, D), jnp.float32),
            pltpu.VMEM((CHUNK, D), jnp.float32),
            pltpu.VMEM((CHUNK, D), jnp.float32),
            pltpu.VMEM((CHUNK, D), jnp.float32),
            pltpu.VMEM((CHUNK, D), jnp.float32),
            pltpu.VMEM((CHUNK, D), jnp.float32),
            pltpu.VMEM((B_PER_W,), jnp.float32),
            pltpu.VMEM_SHARED((NUM_REL, D), jnp.float32),
            pltpu.VMEM_SHARED((NUM_REL, D), jnp.float32),
            pltpu.SemaphoreType.DMA,
            pltpu.SemaphoreType.DMA,
            pltpu.SemaphoreType.DMA,
            pltpu.SemaphoreType.DMA,
            pltpu.SemaphoreType.DMA,
        ],
    )(_sc_body)
    return run(h, r, t, entity_re, entity_im, relation_re, relation_im)


def kernel(h, r, t, entity_re, entity_im, relation_re, relation_im):
    return _complex_score(h.astype(jnp.int32), r.astype(jnp.int32),
                          t.astype(jnp.int32), entity_re, entity_im,
                          relation_re, relation_im)


# speedup vs baseline: 1.6733x; 1.1391x over previous
import functools

import jax
import jax.numpy as jnp
from jax import lax
from jax.experimental import pallas as pl
from jax.experimental.pallas import tpu as pltpu
from jax.experimental.pallas import tpu_sc as plsc

NC = 2
NS = 16
NW = NC * NS
L = 16

BATCH = 16384
D = 64
SL = 8
B_PER_W = BATCH // NW
CHUNK = 128
NCHUNK = B_PER_W // CHUNK
NGROUP = CHUNK // L

NUM_REL = 1000
REL_SLABS = NUM_REL // SL
SLABS_PER_TILE = (REL_SLABS + NS - 1) // NS


def _sc_body(h_hbm, r_hbm, t_hbm, ere_hbm, eim_hbm, rre_hbm, rim_hbm,
             out_hbm, idx_h, idx_r, idx_t,
             hr_b, hi_b, tr_b, ti_b, rr_b, ri_b, out_v,
             rre_sp, rim_sp, sem_h, sem_i, sem_t, sem_j, sem_r):
    sid = lax.axis_index("s")
    wid = sid * NC + lax.axis_index("c")
    base = wid * B_PER_W

    rows0 = lax.iota(jnp.int32, L)

    rre_v = rre_hbm.reshape(REL_SLABS, SL, D)
    rim_v = rim_hbm.reshape(REL_SLABS, SL, D)
    rre_sp3 = rre_sp.reshape(REL_SLABS, SL, D)
    rim_sp3 = rim_sp.reshape(REL_SLABS, SL, D)
    for q in range(SLABS_PER_TILE):
        k = sid * SLABS_PER_TILE + q

        @pl.when(k < REL_SLABS)
        def _():
            pltpu.async_copy(rre_v.at[pl.ds(k, 1)], rre_sp3.at[pl.ds(k, 1)],
                             sem_r).wait()
            pltpu.async_copy(rim_v.at[pl.ds(k, 1)], rim_sp3.at[pl.ds(k, 1)],
                             sem_r).wait()

    plsc.subcore_barrier()

    for c in range(NCHUNK):
        off = base + c * CHUNK
        pltpu.sync_copy(h_hbm.at[pl.ds(off, CHUNK)], idx_h)
        pltpu.sync_copy(r_hbm.at[pl.ds(off, CHUNK)], idx_r)
        pltpu.sync_copy(t_hbm.at[pl.ds(off, CHUNK)], idx_t)

        rel_copies = [
            pltpu.async_copy(rre_sp.at[idx_r], rr_b, sem_r),
            pltpu.async_copy(rim_sp.at[idx_r], ri_b, sem_r),
        ]

        def group_dma(g, carry):
            gs = pl.ds(g * L, L)
            ihv = idx_h[gs]
            itv = idx_t[gs]
            for j in range(L):
                m = rows0 == j
                ih = jnp.sum(jnp.where(m, ihv, 0))
                it = jnp.sum(jnp.where(m, itv, 0))
                dst = pl.ds(g * L + j, 1)
                pltpu.async_copy(ere_hbm.at[pl.ds(ih, 1)], hr_b.at[dst],
                                 sem_h)
                pltpu.async_copy(eim_hbm.at[pl.ds(ih, 1)], hi_b.at[dst],
                                 sem_i)
                pltpu.async_copy(ere_hbm.at[pl.ds(it, 1)], tr_b.at[dst],
                                 sem_t)
                pltpu.async_copy(eim_hbm.at[pl.ds(it, 1)], ti_b.at[dst],
                                 sem_j)
            return carry

        lax.fori_loop(0, NGROUP, group_dma, 0)

        pltpu.make_async_copy(ere_hbm.at[pl.ds(0, CHUNK)], hr_b, sem_h).wait()
        pltpu.make_async_copy(ere_hbm.at[pl.ds(0, CHUNK)], hi_b, sem_i).wait()
        pltpu.make_async_copy(ere_hbm.at[pl.ds(0, CHUNK)], tr_b, sem_t).wait()
        pltpu.make_async_copy(ere_hbm.at[pl.ds(0, CHUNK)], ti_b, sem_j).wait()
        for cp in rel_copies:
            cp.wait()

        def group_compute(g, carry):
            def row_step(j, out_vec):
                i = g * L + j
                acc = jnp.zeros((L,), jnp.float32)
                for s in range(D // L):
                    sl = pl.ds(s * L, L)
                    hr = hr_b[i, sl]
                    hi = hi_b[i, sl]
                    tr = tr_b[i, sl]
                    ti = ti_b[i, sl]
                    rr = rr_b[i, sl]
                    ri = ri_b[i, sl]
                    a = hr * rr - hi * ri
                    b = hi * rr + hr * ri
                    acc = acc + a * tr + b * ti
                return jnp.where(rows0 == j, jnp.sum(acc), out_vec)

            out_vec = lax.fori_loop(0, L, row_step,
                                    jnp.zeros((L,), jnp.float32))
            out_v[pl.ds(c * CHUNK + g * L, L)] = out_vec
            return carry

        lax.fori_loop(0, NGROUP, group_compute, 0)

    pltpu.sync_copy(out_v, out_hbm.at[pl.ds(base, B_PER_W)])


@jax.jit
def _complex_score(h, r, t, entity_re, entity_im, relation_re, relation_im):
    mesh = plsc.VectorSubcoreMesh(core_axis_name="c", subcore_axis_name="s")
    run = functools.partial(
        pl.kernel,
        out_type=jax.ShapeDtypeStruct((BATCH,), jnp.float32),
        mesh=mesh,
        compiler_params=pltpu.CompilerParams(needs_layout_passes=False),
        scratch_types=[
            pltpu.VMEM((CHUNK,), jnp.int32),
            pltpu.VMEM((CHUNK,), jnp.int32),
            pltpu.VMEM((CHUNK,), jnp.int32),
            pltpu.VMEM((CHUNK, D), jnp.float32),
            pltpu.VMEM((CHUNK, D), jnp.float32),
            pltpu.VMEM((CHUNK, D), jnp.float32),
            pltpu.VMEM((CHUNK, D), jnp.float32),
            pltpu.VMEM((CHUNK, D), jnp.float32),
            pltpu.VMEM((CHUNK, D), jnp.float32),
            pltpu.VMEM((B_PER_W,), jnp.float32),
            pltpu.VMEM_SHARED((NUM_REL, D), jnp.float32),
            pltpu.VMEM_SHARED((NUM_REL, D), jnp.float32),
            pltpu.SemaphoreType.DMA,
            pltpu.SemaphoreType.DMA,
            pltpu.SemaphoreType.DMA,
            pltpu.SemaphoreType.DMA,
            pltpu.SemaphoreType.DMA,
        ],
    )(_sc_body)
    return run(h, r, t, entity_re, entity_im, relation_re, relation_im)


def kernel(h, r, t, entity_re, entity_im, relation_re, relation_im):
    return _complex_score(h.astype(jnp.int32), r.astype(jnp.int32),
                          t.astype(jnp.int32), entity_re, entity_im,
                          relation_re, relation_im)
